# baseline (speedup 1.0000x reference)
import functools

import jax
import jax.numpy as jnp
from jax import lax
from jax.experimental import pallas as pl
from jax.experimental.pallas import tpu as pltpu
from jax.experimental.pallas import tpu_sc as plsc

N = 10000
NP = 10240
E = 320000
H = 128
NC, NS = 2, 16
NW = NC * NS
C = 80
NCH = E // C
NCHW = NCH // NW
EPW = E // NW
NB = 4
H2 = H // 2
BN = 1024
BE = 2560

_mesh = plsc.VectorSubcoreMesh(
    core_axis_name="c", subcore_axis_name="s", num_cores=NC, num_subcores=NS)
_sc_params = pltpu.CompilerParams(needs_layout_passes=False)
_sc_params_lin = pltpu.CompilerParams(
    needs_layout_passes=False, use_tc_tiling_on_sc=False)


def _wid():
    return lax.axis_index("c") * NS + lax.axis_index("s")


def _deg_body(dst_hbm, out_hbm, deg_v, idx_v, sem):
    wid = _wid()
    idx_dma = pltpu.async_copy(dst_hbm.at[pl.ds(wid * EPW, EPW)], idx_v, sem)

    def zero(i, carry):
        deg_v[pl.ds(i * 16, 16)] = jnp.zeros((16,), jnp.float32)
        return carry
    lax.fori_loop(0, NP // 16, zero, 0)
    idx_dma.wait()

    ones = jnp.ones((16,), jnp.float32)

    def sub(j, carry):
        idx = idx_v[pl.ds(j * 16, 16)]
        plsc.addupdate_scatter(deg_v, [idx], ones)
        return carry
    lax.fori_loop(0, EPW // 16, sub, 0)
    pltpu.sync_copy(deg_v, out_hbm.at[wid])


_deg_call = pl.kernel(
    _deg_body,
    out_type=jax.ShapeDtypeStruct((NW, NP), jnp.float32),
    mesh=_mesh,
    compiler_params=_sc_params,
    scratch_types=[
        pltpu.VMEM((NP,), jnp.float32),
        pltpu.VMEM((EPW,), jnp.int32),
        pltpu.SemaphoreType.DMA,
    ],
)


NBS = 3


def _seg_body(g_hbm, src_hbm, dst_hbm, zero_hbm, p_hbm,
              idx_s, idx_d, rows, acc, gsem, ssem):
    cid = lax.axis_index("c")
    sid = lax.axis_index("s")
    wid = cid * NS + sid
    rpw = NP // NS

    pltpu.sync_copy(zero_hbm.at[pl.ds(sid * rpw, rpw)],
                    acc.at[pl.ds(sid * rpw, rpw)])
    plsc.subcore_barrier()

    def fire_gather(i, b):
        base = wid * EPW + i * C
        pltpu.sync_copy(src_hbm.at[pl.ds(base, C)], idx_s.at[b])
        pltpu.sync_copy(dst_hbm.at[pl.ds(base, C)], idx_d.at[b])
        pltpu.async_copy(g_hbm.at[idx_s.at[b]], rows.at[b], gsem)

    def wait_gather():
        pltpu.make_async_copy(g_hbm.at[idx_s.at[0]], rows.at[0], gsem).wait()

    def fire_scatter(b):
        pltpu.async_copy(rows.at[b], acc.at[idx_d.at[b]], ssem, add=True)

    def wait_scatter():
        pltpu.make_async_copy(rows.at[0], acc.at[idx_d.at[0]], ssem).wait()

    for i0 in range(NBS - 1):
        fire_gather(i0, i0)

    def grp(k, carry):
        for b in range(NBS):
            i = NBS * k + b

            @pl.when(i >= 1)
            def _():
                wait_scatter()

            @pl.when(i + NBS - 1 < NCHW)
            def _():
                fire_gather(i + NBS - 1, (b + NBS - 1) % NBS)
            wait_gather()
            fire_scatter(b)
        return carry
    lax.fori_loop(0, NCHW // NBS, grp, 0)
    for i in range(NCHW - NCHW % NBS, NCHW):
        wait_scatter()
        wait_gather()
        fire_scatter(i % NBS)
    wait_scatter()

    plsc.subcore_barrier()
    pltpu.sync_copy(acc.at[pl.ds(sid * rpw, rpw)],
                    p_hbm.at[cid].at[pl.ds(sid * rpw, rpw)])


_seg_call = pl.kernel(
    _seg_body,
    out_type=jax.ShapeDtypeStruct((NC, NP, H), jnp.float32),
    mesh=_mesh,
    compiler_params=_sc_params,
    scratch_types=[
        pltpu.VMEM((NBS, C), jnp.int32),
        pltpu.VMEM((NBS, C), jnp.int32),
        pltpu.VMEM((NBS, C, H), jnp.float32),
        pltpu.VMEM_SHARED((NP, H), jnp.float32),
        pltpu.SemaphoreType.DMA,
        pltpu.SemaphoreType.DMA,
    ],
)


def _cls_body(a_hbm, b_hbm, row2_hbm, col2_hbm, s_hbm,
              idx_r, idx_c, rows_a, rows_b, gsem, wsem):
    wid = _wid()
    pltpu.sync_copy(row2_hbm.at[wid], idx_r)
    pltpu.sync_copy(col2_hbm.at[wid], idx_c)

    def fire_gathers(i, b):
        pltpu.async_copy(a_hbm.at[idx_r.at[i]], rows_a.at[b], gsem)
        pltpu.async_copy(b_hbm.at[idx_c.at[i]], rows_b.at[b], gsem)

    def wait_gathers():
        pltpu.make_async_copy(a_hbm.at[idx_r.at[0]], rows_a.at[0], gsem).wait()
        pltpu.make_async_copy(b_hbm.at[idx_c.at[0]], rows_b.at[0], gsem).wait()

    def fire_store(i, b):
        base = (wid * NCHW + i) * C
        pltpu.async_copy(rows_a.at[b], s_hbm.at[pl.ds(base, C)], wsem)

    def wait_store():
        pltpu.make_async_copy(rows_a.at[0], s_hbm.at[pl.ds(0, C)], wsem).wait()

    def vadd(b):
        def add_row(r, c2):
            for col in range(H // 16):
                sl = pl.ds(col * 16, 16)
                rows_a[b, r, sl] = rows_a[b, r, sl] + rows_b[b, r, sl]
            return c2
        lax.fori_loop(0, C, add_row, 0)

    for i0 in range(NB - 1):
        fire_gathers(i0, i0)

    def quad(k, carry):
        for b in range(NB):
            i = NB * k + b

            @pl.when(i >= 1)
            def _():
                wait_store()

            @pl.when(i + NB - 1 < NCHW)
            def _():
                fire_gathers(i + NB - 1, (b + NB - 1) % NB)
            wait_gathers()
            vadd(b)
            fire_store(i, b)
        return carry
    lax.fori_loop(0, NCHW // NB, quad, 0)
    for i in range(NCHW - NCHW % NB, NCHW):
        wait_store()
        wait_gathers()
        vadd(i % NB)
        fire_store(i, i % NB)
    wait_store()


_cls_call = pl.kernel(
    _cls_body,
    out_type=jax.ShapeDtypeStruct((E, H), jnp.float32),
    mesh=_mesh,
    compiler_params=_sc_params,
    scratch_types=[
        pltpu.VMEM((NCHW, C), jnp.int32),
        pltpu.VMEM((NCHW, C), jnp.int32),
        pltpu.VMEM((NB, C, H), jnp.float32),
        pltpu.VMEM((NB, C, H), jnp.float32),
        pltpu.SemaphoreType.DMA,
        pltpu.SemaphoreType.DMA,
    ],
)


def _xw_kernel(x_ref, w_ref, out_ref):
    out_ref[...] = x_ref[...] @ w_ref[...]


def _xw_call(x_pad, W1):
    return pl.pallas_call(
        _xw_kernel,
        grid=(NP // BN,),
        in_specs=[
            pl.BlockSpec((BN, H), lambda i: (i, 0)),
            pl.BlockSpec((H, H), lambda i: (0, 0)),
        ],
        out_specs=pl.BlockSpec((BN, H), lambda i: (i, 0)),
        out_shape=jax.ShapeDtypeStruct((NP, H), jnp.float32),
    )(x_pad, W1)


def _g1_kernel(degp_ref, xw_ref, dinv_ref, g_ref):
    deg = jnp.sum(degp_ref[...], axis=0) + 1.0
    dinv = lax.rsqrt(deg)
    dinv_ref[...] = dinv
    g_ref[...] = xw_ref[...] * dinv[:, None]


def _g1_call(degp, xw):
    return pl.pallas_call(
        _g1_kernel,
        grid=(NP // BN,),
        in_specs=[
            pl.BlockSpec((NW, BN), lambda i: (0, i)),
            pl.BlockSpec((BN, H), lambda i: (i, 0)),
        ],
        out_specs=[
            pl.BlockSpec((BN,), lambda i: (i,)),
            pl.BlockSpec((BN, H), lambda i: (i, 0)),
        ],
        out_shape=[
            jax.ShapeDtypeStruct((NP,), jnp.float32),
            jax.ShapeDtypeStruct((NP, H), jnp.float32),
        ],
    )(degp, xw)


def _comb_kernel(p_ref, g_ref, dinv_ref, b_ref, w_ref, out_ref):
    dinv = dinv_ref[...]
    h = jnp.maximum(
        (p_ref[0] + p_ref[1] + g_ref[...]) * dinv[:, None] + b_ref[...], 0.0)
    out_ref[...] = (h @ w_ref[...]) * dinv[:, None]


def _comb_call(p, g, dinv, b_row, W_next):
    return pl.pallas_call(
        _comb_kernel,
        grid=(NP // BN,),
        in_specs=[
            pl.BlockSpec((NC, BN, H), lambda i: (0, i, 0)),
            pl.BlockSpec((BN, H), lambda i: (i, 0)),
            pl.BlockSpec((BN,), lambda i: (i,)),
            pl.BlockSpec((1, H), lambda i: (0, 0)),
            pl.BlockSpec((H, H), lambda i: (0, 0)),
        ],
        out_specs=pl.BlockSpec((BN, H), lambda i: (i, 0)),
        out_shape=jax.ShapeDtypeStruct((NP, H), jnp.float32),
    )(p, g, dinv, b_row, W_next)


def _post_kernel(p_ref, g_ref, dinv_ref, bf_ref, wa1_ref, ba1_ref,
                 wa2_ref, ba2_ref, w1a_ref, w1b_ref, a_ref, b_out_ref):
    dinv = dinv_ref[...]
    h = jnp.maximum(
        (p_ref[0] + p_ref[1] + g_ref[...]) * dinv[:, None] + bf_ref[...], 0.0)
    t = jnp.maximum(h @ wa1_ref[...] + ba1_ref[...], 0.0)
    att = jax.nn.sigmoid(
        jnp.sum(t * wa2_ref[...], axis=1, keepdims=True) + ba2_ref[...])
    hg = h * att
    a_ref[...] = hg @ w1a_ref[...]
    b_out_ref[...] = hg @ w1b_ref[...]


def _post_call(p, g, dinv, bf_row, Wa1, ba1_row, wa2_row, ba2_11, W1a, W1b):
    return pl.pallas_call(
        _post_kernel,
        grid=(NP // BN,),
        in_specs=[
            pl.BlockSpec((NC, BN, H), lambda i: (0, i, 0)),
            pl.BlockSpec((BN, H), lambda i: (i, 0)),
            pl.BlockSpec((BN,), lambda i: (i,)),
            pl.BlockSpec((1, H), lambda i: (0, 0)),
            pl.BlockSpec((H, H // 2), lambda i: (0, 0)),
            pl.BlockSpec((1, H // 2), lambda i: (0, 0)),
            pl.BlockSpec((1, H // 2), lambda i: (0, 0)),
            pl.BlockSpec((1, 1), lambda i: (0, 0)),
            pl.BlockSpec((H, H), lambda i: (0, 0)),
            pl.BlockSpec((H, H), lambda i: (0, 0)),
        ],
        out_specs=[
            pl.BlockSpec((BN, H), lambda i: (i, 0)),
            pl.BlockSpec((BN, H), lambda i: (i, 0)),
        ],
        out_shape=[
            jax.ShapeDtypeStruct((NP, H), jnp.float32),
            jax.ShapeDtypeStruct((NP, H), jnp.float32),
        ],
    )(p, g, dinv, bf_row, Wa1, ba1_row, wa2_row, ba2_11, W1a, W1b)


def _bdot(a, b):
    return lax.dot_general(
        a.astype(jnp.bfloat16), b.astype(jnp.bfloat16),
        (((1,), (0,)), ((), ())), preferred_element_type=jnp.float32)


def _edge_kernel(s_ref, ea_ref, we1_ref, be1_ref, we2_ref, be2_ref,
                 wc1c_ref, bc1_ref, wc2_ref, bc2_ref, wc3_ref, bc3_ref,
                 out_ref):
    t = jnp.maximum(ea_ref[...] @ we1_ref[...] + be1_ref[...], 0.0)
    wprime = we2_ref[...] @ wc1c_ref[...]
    cprime = be2_ref[...] @ wc1c_ref[...] + bc1_ref[...]
    z = jnp.maximum(_bdot(t, wprime) + s_ref[...] + cprime, 0.0)
    z2 = jnp.maximum(_bdot(z, wc2_ref[...]) + bc2_ref[...], 0.0)
    l0 = jnp.sum(z2 * wc3_ref[0:1, :], axis=1) + bc3_ref[0, 0]
    l1 = jnp.sum(z2 * wc3_ref[1:2, :], axis=1) + bc3_ref[0, 1]
    m = jnp.maximum(l0, l1)
    lse = m + jnp.log(jnp.exp(l0 - m) + jnp.exp(l1 - m))
    out_ref[0, :] = l0 - lse
    out_ref[1, :] = l1 - lse


def _edge_call(S, ea, We1, be1_row, We2, be2_row, Wc1c, bc1_row,
               Wc2, bc2_row, Wc3, bc3_row):
    full = lambda shape: pl.BlockSpec(shape, lambda i: tuple(0 for _ in shape))
    return pl.pallas_call(
        _edge_kernel,
        grid=(E // BE,),
        in_specs=[
            pl.BlockSpec((BE, H), lambda i: (i, 0)),
            pl.BlockSpec((BE, 4), lambda i: (i, 0)),
            full((4, H)),
            full((1, H)),
            full((H, H)),
            full((1, H)),
            full((H, H)),
            full((1, H)),
            full((H, H // 2)),
            full((1, H // 2)),
            full((2, H // 2)),
            full((1, 2)),
        ],
        out_specs=pl.BlockSpec((2, BE), lambda i: (0, i)),
        out_shape=jax.ShapeDtypeStruct((2, E), jnp.float32),
    )(S, ea, We1, be1_row, We2, be2_row, Wc1c, bc1_row,
      Wc2, bc2_row, Wc3, bc3_row)


def kernel(x, edge_index, edge_attr, W1, b1, W2, b2, W3, b3, Wf, bf,
           We1, be1, We2, be2, Wa1, ba1, Wa2, ba2,
           Wc1, bc1, Wc2, bc2, Wc3, bc3):
    src = edge_index[0]
    dst = edge_index[1]
    src2 = src.reshape(NW, NCHW, C)
    dst2 = dst.reshape(NW, NCHW, C)
    x_pad = jnp.pad(x, ((0, NP - N), (0, 0)))
    zeros_np = jnp.zeros((NP, H), jnp.float32)

    degp = _deg_call(dst)
    xw = _xw_call(x_pad, W1)
    dinv, g = _g1_call(degp, xw)

    for W_next, b_cur in ((W2, b1), (W3, b2), (Wf, b3)):
        p = _seg_call(g, src, dst, zeros_np)
        g = _comb_call(p, g, dinv, b_cur.reshape(1, H), W_next)

    p = _seg_call(g, src, dst, zeros_np)
    A, B = _post_call(
        p, g, dinv, bf.reshape(1, H), Wa1, ba1.reshape(1, H // 2),
        Wa2.reshape(1, H // 2), ba2.reshape(1, 1),
        Wc1[:H], Wc1[H:2 * H])

    S = _cls_call(A, B, src2, dst2)

    out_t = _edge_call(
        S, edge_attr, We1, be1.reshape(1, H), We2, be2.reshape(1, H),
        Wc1[2 * H:], bc1.reshape(1, H), Wc2, bc2.reshape(1, H // 2),
        Wc3.T, bc3.reshape(1, 2))
    return out_t.T


# speedup vs baseline: 1.1899x; 1.1899x over previous
import functools

import jax
import jax.numpy as jnp
from jax import lax
from jax.experimental import pallas as pl
from jax.experimental.pallas import tpu as pltpu
from jax.experimental.pallas import tpu_sc as plsc

N = 10000
NP = 10240
E = 320000
H = 128
NC, NS = 2, 16
NW = NC * NS
C = 80
NCH = E // C
NCHW = NCH // NW
EPW = E // NW
NB = 4
H2 = H // 2
BN = 1024
BE = 3200

_mesh = plsc.VectorSubcoreMesh(
    core_axis_name="c", subcore_axis_name="s", num_cores=NC, num_subcores=NS)
_sc_params = pltpu.CompilerParams(needs_layout_passes=False)
_sc_params_lin = pltpu.CompilerParams(
    needs_layout_passes=False, use_tc_tiling_on_sc=False)


def _wid():
    return lax.axis_index("c") * NS + lax.axis_index("s")


def _deg_body(dst_hbm, out_hbm, deg_v, idx_v, sem):
    wid = _wid()
    idx_dma = pltpu.async_copy(dst_hbm.at[pl.ds(wid * EPW, EPW)], idx_v, sem)

    def zero(i, carry):
        deg_v[pl.ds(i * 16, 16)] = jnp.zeros((16,), jnp.float32)
        return carry
    lax.fori_loop(0, NP // 16, zero, 0)
    idx_dma.wait()

    ones = jnp.ones((16,), jnp.float32)

    def sub(j, carry):
        idx = idx_v[pl.ds(j * 16, 16)]
        plsc.addupdate_scatter(deg_v, [idx], ones)
        return carry
    lax.fori_loop(0, EPW // 16, sub, 0)
    pltpu.sync_copy(deg_v, out_hbm.at[wid])


_deg_call = pl.kernel(
    _deg_body,
    out_type=jax.ShapeDtypeStruct((NW, NP), jnp.float32),
    mesh=_mesh,
    compiler_params=_sc_params,
    scratch_types=[
        pltpu.VMEM((NP,), jnp.float32),
        pltpu.VMEM((EPW,), jnp.int32),
        pltpu.SemaphoreType.DMA,
    ],
)


NBS = 3


def _seg_body(g_hbm, src_hbm, dst_hbm, zero_hbm, p_hbm,
              idx_s, idx_d, rows, acc, gsem, ssem):
    cid = lax.axis_index("c")
    sid = lax.axis_index("s")
    wid = cid * NS + sid
    rpw = NP // NS

    pltpu.sync_copy(zero_hbm.at[pl.ds(sid * rpw, rpw)],
                    acc.at[pl.ds(sid * rpw, rpw)])
    plsc.subcore_barrier()

    def fire_gather(i, b):
        base = wid * EPW + i * C
        pltpu.sync_copy(src_hbm.at[pl.ds(base, C)], idx_s.at[b])
        pltpu.sync_copy(dst_hbm.at[pl.ds(base, C)], idx_d.at[b])
        pltpu.async_copy(g_hbm.at[idx_s.at[b]], rows.at[b], gsem)

    def wait_gather():
        pltpu.make_async_copy(g_hbm.at[idx_s.at[0]], rows.at[0], gsem).wait()

    def fire_scatter(b):
        pltpu.async_copy(rows.at[b], acc.at[idx_d.at[b]], ssem, add=True)

    def wait_scatter():
        pltpu.make_async_copy(rows.at[0], acc.at[idx_d.at[0]], ssem).wait()

    for i0 in range(NBS - 1):
        fire_gather(i0, i0)

    def grp(k, carry):
        for b in range(NBS):
            i = NBS * k + b

            @pl.when(i >= 1)
            def _():
                wait_scatter()

            @pl.when(i + NBS - 1 < NCHW)
            def _():
                fire_gather(i + NBS - 1, (b + NBS - 1) % NBS)
            wait_gather()
            fire_scatter(b)
        return carry
    lax.fori_loop(0, NCHW // NBS, grp, 0)
    for i in range(NCHW - NCHW % NBS, NCHW):
        wait_scatter()
        wait_gather()
        fire_scatter(i % NBS)
    wait_scatter()

    plsc.subcore_barrier()
    pltpu.sync_copy(acc.at[pl.ds(sid * rpw, rpw)],
                    p_hbm.at[cid].at[pl.ds(sid * rpw, rpw)])


_seg_call = pl.kernel(
    _seg_body,
    out_type=jax.ShapeDtypeStruct((NC, NP, H), jnp.float32),
    mesh=_mesh,
    compiler_params=_sc_params,
    scratch_types=[
        pltpu.VMEM((NBS, C), jnp.int32),
        pltpu.VMEM((NBS, C), jnp.int32),
        pltpu.VMEM((NBS, C, H), jnp.float32),
        pltpu.VMEM_SHARED((NP, H), jnp.float32),
        pltpu.SemaphoreType.DMA,
        pltpu.SemaphoreType.DMA,
    ],
)


def _cls_body(a_hbm, b_hbm, row2_hbm, col2_hbm, s_hbm,
              idx_r, idx_c, rows_a, rows_b, gsem, wsem):
    wid = _wid()
    pltpu.sync_copy(row2_hbm.at[wid], idx_r)
    pltpu.sync_copy(col2_hbm.at[wid], idx_c)

    def fire_gathers(i, b):
        pltpu.async_copy(a_hbm.at[idx_r.at[i]], rows_a.at[b], gsem)
        pltpu.async_copy(b_hbm.at[idx_c.at[i]], rows_b.at[b], gsem)

    def wait_gathers():
        pltpu.make_async_copy(a_hbm.at[idx_r.at[0]], rows_a.at[0], gsem).wait()
        pltpu.make_async_copy(b_hbm.at[idx_c.at[0]], rows_b.at[0], gsem).wait()

    def fire_store(i, b):
        base = (wid * NCHW + i) * C
        pltpu.async_copy(rows_a.at[b], s_hbm.at[pl.ds(base, C)], wsem)

    def wait_store():
        pltpu.make_async_copy(rows_a.at[0], s_hbm.at[pl.ds(0, C)], wsem).wait()

    def vadd(b):
        def add_row(r, c2):
            for col in range(H // 16):
                sl = pl.ds(col * 16, 16)
                rows_a[b, r, sl] = rows_a[b, r, sl] + rows_b[b, r, sl]
            return c2
        lax.fori_loop(0, C, add_row, 0)

    for i0 in range(NB - 1):
        fire_gathers(i0, i0)

    def quad(k, carry):
        for b in range(NB):
            i = NB * k + b

            @pl.when(i >= 1)
            def _():
                wait_store()

            @pl.when(i + NB - 1 < NCHW)
            def _():
                fire_gathers(i + NB - 1, (b + NB - 1) % NB)
            wait_gathers()
            vadd(b)
            fire_store(i, b)
        return carry
    lax.fori_loop(0, NCHW // NB, quad, 0)
    for i in range(NCHW - NCHW % NB, NCHW):
        wait_store()
        wait_gathers()
        vadd(i % NB)
        fire_store(i, i % NB)
    wait_store()


_cls_call = pl.kernel(
    _cls_body,
    out_type=jax.ShapeDtypeStruct((E, H), jnp.float32),
    mesh=_mesh,
    compiler_params=_sc_params,
    scratch_types=[
        pltpu.VMEM((NCHW, C), jnp.int32),
        pltpu.VMEM((NCHW, C), jnp.int32),
        pltpu.VMEM((NB, C, H), jnp.float32),
        pltpu.VMEM((NB, C, H), jnp.float32),
        pltpu.SemaphoreType.DMA,
        pltpu.SemaphoreType.DMA,
    ],
)


def _xw_kernel(x_ref, w_ref, out_ref):
    out_ref[...] = x_ref[...] @ w_ref[...]


def _xw_call(x_pad, W1):
    return pl.pallas_call(
        _xw_kernel,
        grid=(NP // BN,),
        in_specs=[
            pl.BlockSpec((BN, H), lambda i: (i, 0)),
            pl.BlockSpec((H, H), lambda i: (0, 0)),
        ],
        out_specs=pl.BlockSpec((BN, H), lambda i: (i, 0)),
        out_shape=jax.ShapeDtypeStruct((NP, H), jnp.float32),
    )(x_pad, W1)


def _g1_kernel(degp_ref, xw_ref, dinv_ref, g_ref):
    deg = jnp.sum(degp_ref[...], axis=0) + 1.0
    dinv = lax.rsqrt(deg)
    dinv_ref[...] = dinv
    g_ref[...] = xw_ref[...] * dinv[:, None]


def _g1_call(degp, xw):
    return pl.pallas_call(
        _g1_kernel,
        grid=(NP // BN,),
        in_specs=[
            pl.BlockSpec((NW, BN), lambda i: (0, i)),
            pl.BlockSpec((BN, H), lambda i: (i, 0)),
        ],
        out_specs=[
            pl.BlockSpec((BN,), lambda i: (i,)),
            pl.BlockSpec((BN, H), lambda i: (i, 0)),
        ],
        out_shape=[
            jax.ShapeDtypeStruct((NP,), jnp.float32),
            jax.ShapeDtypeStruct((NP, H), jnp.float32),
        ],
    )(degp, xw)


def _comb_kernel(p_ref, g_ref, dinv_ref, b_ref, w_ref, out_ref):
    dinv = dinv_ref[...]
    h = jnp.maximum(
        (p_ref[0] + p_ref[1] + g_ref[...]) * dinv[:, None] + b_ref[...], 0.0)
    out_ref[...] = (h @ w_ref[...]) * dinv[:, None]


def _comb_call(p, g, dinv, b_row, W_next):
    return pl.pallas_call(
        _comb_kernel,
        grid=(NP // BN,),
        in_specs=[
            pl.BlockSpec((NC, BN, H), lambda i: (0, i, 0)),
            pl.BlockSpec((BN, H), lambda i: (i, 0)),
            pl.BlockSpec((BN,), lambda i: (i,)),
            pl.BlockSpec((1, H), lambda i: (0, 0)),
            pl.BlockSpec((H, H), lambda i: (0, 0)),
        ],
        out_specs=pl.BlockSpec((BN, H), lambda i: (i, 0)),
        out_shape=jax.ShapeDtypeStruct((NP, H), jnp.float32),
    )(p, g, dinv, b_row, W_next)


def _post_kernel(p_ref, g_ref, dinv_ref, bf_ref, wa1_ref, ba1_ref,
                 wa2_ref, ba2_ref, w1a_ref, w1b_ref, a_ref, b_out_ref):
    dinv = dinv_ref[...]
    h = jnp.maximum(
        (p_ref[0] + p_ref[1] + g_ref[...]) * dinv[:, None] + bf_ref[...], 0.0)
    t = jnp.maximum(h @ wa1_ref[...] + ba1_ref[...], 0.0)
    att = jax.nn.sigmoid(
        jnp.sum(t * wa2_ref[...], axis=1, keepdims=True) + ba2_ref[...])
    hg = h * att
    a_ref[...] = hg @ w1a_ref[...]
    b_out_ref[...] = hg @ w1b_ref[...]


def _post_call(p, g, dinv, bf_row, Wa1, ba1_row, wa2_row, ba2_11, W1a, W1b):
    return pl.pallas_call(
        _post_kernel,
        grid=(NP // BN,),
        in_specs=[
            pl.BlockSpec((NC, BN, H), lambda i: (0, i, 0)),
            pl.BlockSpec((BN, H), lambda i: (i, 0)),
            pl.BlockSpec((BN,), lambda i: (i,)),
            pl.BlockSpec((1, H), lambda i: (0, 0)),
            pl.BlockSpec((H, H // 2), lambda i: (0, 0)),
            pl.BlockSpec((1, H // 2), lambda i: (0, 0)),
            pl.BlockSpec((1, H // 2), lambda i: (0, 0)),
            pl.BlockSpec((1, 1), lambda i: (0, 0)),
            pl.BlockSpec((H, H), lambda i: (0, 0)),
            pl.BlockSpec((H, H), lambda i: (0, 0)),
        ],
        out_specs=[
            pl.BlockSpec((BN, H), lambda i: (i, 0)),
            pl.BlockSpec((BN, H), lambda i: (i, 0)),
        ],
        out_shape=[
            jax.ShapeDtypeStruct((NP, H), jnp.float32),
            jax.ShapeDtypeStruct((NP, H), jnp.float32),
        ],
    )(p, g, dinv, bf_row, Wa1, ba1_row, wa2_row, ba2_11, W1a, W1b)


def _bdot(a, b):
    return lax.dot_general(
        a.astype(jnp.bfloat16), b.astype(jnp.bfloat16),
        (((1,), (0,)), ((), ())), preferred_element_type=jnp.float32)


def _edge_kernel(s_ref, ea_ref, we1_ref, be1_ref, we2_ref, be2_ref,
                 wc1c_ref, bc1_ref, wc2_ref, bc2_ref, wc3_ref, bc3_ref,
                 out_ref):
    t = jnp.maximum(ea_ref[...] @ we1_ref[...] + be1_ref[...], 0.0)
    wprime = we2_ref[...] @ wc1c_ref[...]
    cprime = be2_ref[...] @ wc1c_ref[...] + bc1_ref[...]
    z = jnp.maximum(_bdot(t, wprime) + s_ref[...] + cprime, 0.0)
    z2 = jnp.maximum(_bdot(z, wc2_ref[...]) + bc2_ref[...], 0.0)
    logits = z2 @ wc3_ref[...] + bc3_ref[...]
    m = jnp.max(logits, axis=1, keepdims=True)
    lse = m + jnp.log(jnp.sum(jnp.exp(logits - m), axis=1, keepdims=True))
    out_ref[...] = logits - lse


def _edge_call(S, ea, We1, be1_row, We2, be2_row, Wc1c, bc1_row,
               Wc2, bc2_row, Wc3, bc3_row):
    full = lambda shape: pl.BlockSpec(shape, lambda i: tuple(0 for _ in shape))
    return pl.pallas_call(
        _edge_kernel,
        grid=(E // BE,),
        in_specs=[
            pl.BlockSpec((BE, H), lambda i: (i, 0)),
            pl.BlockSpec((BE, 4), lambda i: (i, 0)),
            full((4, H)),
            full((1, H)),
            full((H, H)),
            full((1, H)),
            full((H, H)),
            full((1, H)),
            full((H, H // 2)),
            full((1, H // 2)),
            full((H // 2, 2)),
            full((1, 2)),
        ],
        out_specs=pl.BlockSpec((BE, 2), lambda i: (i, 0)),
        out_shape=jax.ShapeDtypeStruct((E, 2), jnp.float32),
    )(S, ea, We1, be1_row, We2, be2_row, Wc1c, bc1_row,
      Wc2, bc2_row, Wc3, bc3_row)


def kernel(x, edge_index, edge_attr, W1, b1, W2, b2, W3, b3, Wf, bf,
           We1, be1, We2, be2, Wa1, ba1, Wa2, ba2,
           Wc1, bc1, Wc2, bc2, Wc3, bc3):
    src = edge_index[0]
    dst = edge_index[1]
    src2 = src.reshape(NW, NCHW, C)
    dst2 = dst.reshape(NW, NCHW, C)
    x_pad = jnp.pad(x, ((0, NP - N), (0, 0)))
    zeros_np = jnp.zeros((NP, H), jnp.float32)

    degp = _deg_call(dst)
    xw = _xw_call(x_pad, W1)
    dinv, g = _g1_call(degp, xw)

    for W_next, b_cur in ((W2, b1), (W3, b2), (Wf, b3)):
        p = _seg_call(g, src, dst, zeros_np)
        g = _comb_call(p, g, dinv, b_cur.reshape(1, H), W_next)

    p = _seg_call(g, src, dst, zeros_np)
    A, B = _post_call(
        p, g, dinv, bf.reshape(1, H), Wa1, ba1.reshape(1, H // 2),
        Wa2.reshape(1, H // 2), ba2.reshape(1, 1),
        Wc1[:H], Wc1[H:2 * H])

    S = _cls_call(A, B, src2, dst2)

    return _edge_call(
        S, edge_attr, We1, be1.reshape(1, H), We2, be2.reshape(1, H),
        Wc1[2 * H:], bc1.reshape(1, H), Wc2, bc2.reshape(1, H // 2),
        Wc3, bc3.reshape(1, 2))


# speedup vs baseline: 1.2237x; 1.0284x over previous
import functools

import jax
import jax.numpy as jnp
from jax import lax
from jax.experimental import pallas as pl
from jax.experimental.pallas import tpu as pltpu
from jax.experimental.pallas import tpu_sc as plsc

N = 10000
NP = 10240
E = 320000
H = 128
NC, NS = 2, 16
NW = NC * NS
C = 80
NCH = E // C
NCHW = NCH // NW
EPW = E // NW
NB = 4
H2 = H // 2
BN = 1024
BE = 6400

_mesh = plsc.VectorSubcoreMesh(
    core_axis_name="c", subcore_axis_name="s", num_cores=NC, num_subcores=NS)
_sc_params = pltpu.CompilerParams(needs_layout_passes=False)
_sc_params_lin = pltpu.CompilerParams(
    needs_layout_passes=False, use_tc_tiling_on_sc=False)


def _wid():
    return lax.axis_index("c") * NS + lax.axis_index("s")


def _deg_body(dst_hbm, out_hbm, deg_v, idx_v, sem):
    wid = _wid()
    idx_dma = pltpu.async_copy(dst_hbm.at[pl.ds(wid * EPW, EPW)], idx_v, sem)

    def zero(i, carry):
        deg_v[pl.ds(i * 16, 16)] = jnp.zeros((16,), jnp.float32)
        return carry
    lax.fori_loop(0, NP // 16, zero, 0)
    idx_dma.wait()

    ones = jnp.ones((16,), jnp.float32)

    def sub(j, carry):
        idx = idx_v[pl.ds(j * 16, 16)]
        plsc.addupdate_scatter(deg_v, [idx], ones)
        return carry
    lax.fori_loop(0, EPW // 16, sub, 0)
    pltpu.sync_copy(deg_v, out_hbm.at[wid])


_deg_call = pl.kernel(
    _deg_body,
    out_type=jax.ShapeDtypeStruct((NW, NP), jnp.float32),
    mesh=_mesh,
    compiler_params=_sc_params,
    scratch_types=[
        pltpu.VMEM((NP,), jnp.float32),
        pltpu.VMEM((EPW,), jnp.int32),
        pltpu.SemaphoreType.DMA,
    ],
)


NBS = 3


def _seg_body(g_hbm, src_hbm, dst_hbm, zero_hbm, p_hbm,
              idx_s, idx_d, rows, acc, gsem, ssem):
    cid = lax.axis_index("c")
    sid = lax.axis_index("s")
    wid = cid * NS + sid
    rpw = NP // NS

    pltpu.sync_copy(zero_hbm.at[pl.ds(sid * rpw, rpw)],
                    acc.at[pl.ds(sid * rpw, rpw)])
    plsc.subcore_barrier()

    def fire_gather(i, b):
        base = wid * EPW + i * C
        pltpu.sync_copy(src_hbm.at[pl.ds(base, C)], idx_s.at[b])
        pltpu.sync_copy(dst_hbm.at[pl.ds(base, C)], idx_d.at[b])
        pltpu.async_copy(g_hbm.at[idx_s.at[b]], rows.at[b], gsem)

    def wait_gather():
        pltpu.make_async_copy(g_hbm.at[idx_s.at[0]], rows.at[0], gsem).wait()

    def fire_scatter(b):
        pltpu.async_copy(rows.at[b], acc.at[idx_d.at[b]], ssem, add=True)

    def wait_scatter():
        pltpu.make_async_copy(rows.at[0], acc.at[idx_d.at[0]], ssem).wait()

    for i0 in range(NBS - 1):
        fire_gather(i0, i0)

    def grp(k, carry):
        for b in range(NBS):
            i = NBS * k + b

            @pl.when(i >= 1)
            def _():
                wait_scatter()

            @pl.when(i + NBS - 1 < NCHW)
            def _():
                fire_gather(i + NBS - 1, (b + NBS - 1) % NBS)
            wait_gather()
            fire_scatter(b)
        return carry
    lax.fori_loop(0, NCHW // NBS, grp, 0)
    for i in range(NCHW - NCHW % NBS, NCHW):
        wait_scatter()
        wait_gather()
        fire_scatter(i % NBS)
    wait_scatter()

    plsc.subcore_barrier()
    pltpu.sync_copy(acc.at[pl.ds(sid * rpw, rpw)],
                    p_hbm.at[cid].at[pl.ds(sid * rpw, rpw)])


_seg_call = pl.kernel(
    _seg_body,
    out_type=jax.ShapeDtypeStruct((NC, NP, H), jnp.float32),
    mesh=_mesh,
    compiler_params=_sc_params,
    scratch_types=[
        pltpu.VMEM((NBS, C), jnp.int32),
        pltpu.VMEM((NBS, C), jnp.int32),
        pltpu.VMEM((NBS, C, H), jnp.float32),
        pltpu.VMEM_SHARED((NP, H), jnp.float32),
        pltpu.SemaphoreType.DMA,
        pltpu.SemaphoreType.DMA,
    ],
)


def _cls_body(a_hbm, b_hbm, row2_hbm, col2_hbm, s_hbm,
              idx_r, idx_c, rows_a, rows_b, gsem, wsem):
    wid = _wid()
    pltpu.sync_copy(row2_hbm.at[wid], idx_r)
    pltpu.sync_copy(col2_hbm.at[wid], idx_c)

    def fire_gathers(i, b):
        pltpu.async_copy(a_hbm.at[idx_r.at[i]], rows_a.at[b], gsem)
        pltpu.async_copy(b_hbm.at[idx_c.at[i]], rows_b.at[b], gsem)

    def wait_gathers():
        pltpu.make_async_copy(a_hbm.at[idx_r.at[0]], rows_a.at[0], gsem).wait()
        pltpu.make_async_copy(b_hbm.at[idx_c.at[0]], rows_b.at[0], gsem).wait()

    def fire_store(i, b):
        base = (wid * NCHW + i) * C
        pltpu.async_copy(rows_a.at[b], s_hbm.at[pl.ds(base, C)], wsem)

    def wait_store():
        pltpu.make_async_copy(rows_a.at[0], s_hbm.at[pl.ds(0, C)], wsem).wait()

    def vadd(b):
        def add_row(r, c2):
            for col in range(H // 16):
                sl = pl.ds(col * 16, 16)
                rows_a[b, r, sl] = rows_a[b, r, sl] + rows_b[b, r, sl]
            return c2
        lax.fori_loop(0, C, add_row, 0)

    for i0 in range(NB - 1):
        fire_gathers(i0, i0)

    def quad(k, carry):
        for b in range(NB):
            i = NB * k + b

            @pl.when(i >= 1)
            def _():
                wait_store()

            @pl.when(i + NB - 1 < NCHW)
            def _():
                fire_gathers(i + NB - 1, (b + NB - 1) % NB)
            wait_gathers()
            vadd(b)
            fire_store(i, b)
        return carry
    lax.fori_loop(0, NCHW // NB, quad, 0)
    for i in range(NCHW - NCHW % NB, NCHW):
        wait_store()
        wait_gathers()
        vadd(i % NB)
        fire_store(i, i % NB)
    wait_store()


_cls_call = pl.kernel(
    _cls_body,
    out_type=jax.ShapeDtypeStruct((E, H), jnp.float32),
    mesh=_mesh,
    compiler_params=_sc_params,
    scratch_types=[
        pltpu.VMEM((NCHW, C), jnp.int32),
        pltpu.VMEM((NCHW, C), jnp.int32),
        pltpu.VMEM((NB, C, H), jnp.float32),
        pltpu.VMEM((NB, C, H), jnp.float32),
        pltpu.SemaphoreType.DMA,
        pltpu.SemaphoreType.DMA,
    ],
)


def _xw_kernel(x_ref, w_ref, out_ref):
    out_ref[...] = x_ref[...] @ w_ref[...]


def _xw_call(x_pad, W1):
    return pl.pallas_call(
        _xw_kernel,
        grid=(NP // BN,),
        in_specs=[
            pl.BlockSpec((BN, H), lambda i: (i, 0)),
            pl.BlockSpec((H, H), lambda i: (0, 0)),
        ],
        out_specs=pl.BlockSpec((BN, H), lambda i: (i, 0)),
        out_shape=jax.ShapeDtypeStruct((NP, H), jnp.float32),
    )(x_pad, W1)


def _g1_kernel(degp_ref, xw_ref, dinv_ref, g_ref):
    deg = jnp.sum(degp_ref[...], axis=0) + 1.0
    dinv = lax.rsqrt(deg)
    dinv_ref[...] = dinv
    g_ref[...] = xw_ref[...] * dinv[:, None]


def _g1_call(degp, xw):
    return pl.pallas_call(
        _g1_kernel,
        grid=(NP // BN,),
        in_specs=[
            pl.BlockSpec((NW, BN), lambda i: (0, i)),
            pl.BlockSpec((BN, H), lambda i: (i, 0)),
        ],
        out_specs=[
            pl.BlockSpec((BN,), lambda i: (i,)),
            pl.BlockSpec((BN, H), lambda i: (i, 0)),
        ],
        out_shape=[
            jax.ShapeDtypeStruct((NP,), jnp.float32),
            jax.ShapeDtypeStruct((NP, H), jnp.float32),
        ],
    )(degp, xw)


def _comb_kernel(p_ref, g_ref, dinv_ref, b_ref, w_ref, out_ref):
    dinv = dinv_ref[...]
    h = jnp.maximum(
        (p_ref[0] + p_ref[1] + g_ref[...]) * dinv[:, None] + b_ref[...], 0.0)
    out_ref[...] = (h @ w_ref[...]) * dinv[:, None]


def _comb_call(p, g, dinv, b_row, W_next):
    return pl.pallas_call(
        _comb_kernel,
        grid=(NP // BN,),
        in_specs=[
            pl.BlockSpec((NC, BN, H), lambda i: (0, i, 0)),
            pl.BlockSpec((BN, H), lambda i: (i, 0)),
            pl.BlockSpec((BN,), lambda i: (i,)),
            pl.BlockSpec((1, H), lambda i: (0, 0)),
            pl.BlockSpec((H, H), lambda i: (0, 0)),
        ],
        out_specs=pl.BlockSpec((BN, H), lambda i: (i, 0)),
        out_shape=jax.ShapeDtypeStruct((NP, H), jnp.float32),
    )(p, g, dinv, b_row, W_next)


def _post_kernel(p_ref, g_ref, dinv_ref, bf_ref, wa1_ref, ba1_ref,
                 wa2_ref, ba2_ref, w1a_ref, w1b_ref, a_ref, b_out_ref):
    dinv = dinv_ref[...]
    h = jnp.maximum(
        (p_ref[0] + p_ref[1] + g_ref[...]) * dinv[:, None] + bf_ref[...], 0.0)
    t = jnp.maximum(h @ wa1_ref[...] + ba1_ref[...], 0.0)
    att = jax.nn.sigmoid(
        jnp.sum(t * wa2_ref[...], axis=1, keepdims=True) + ba2_ref[...])
    hg = h * att
    a_ref[...] = hg @ w1a_ref[...]
    b_out_ref[...] = hg @ w1b_ref[...]


def _post_call(p, g, dinv, bf_row, Wa1, ba1_row, wa2_row, ba2_11, W1a, W1b):
    return pl.pallas_call(
        _post_kernel,
        grid=(NP // BN,),
        in_specs=[
            pl.BlockSpec((NC, BN, H), lambda i: (0, i, 0)),
            pl.BlockSpec((BN, H), lambda i: (i, 0)),
            pl.BlockSpec((BN,), lambda i: (i,)),
            pl.BlockSpec((1, H), lambda i: (0, 0)),
            pl.BlockSpec((H, H // 2), lambda i: (0, 0)),
            pl.BlockSpec((1, H // 2), lambda i: (0, 0)),
            pl.BlockSpec((1, H // 2), lambda i: (0, 0)),
            pl.BlockSpec((1, 1), lambda i: (0, 0)),
            pl.BlockSpec((H, H), lambda i: (0, 0)),
            pl.BlockSpec((H, H), lambda i: (0, 0)),
        ],
        out_specs=[
            pl.BlockSpec((BN, H), lambda i: (i, 0)),
            pl.BlockSpec((BN, H), lambda i: (i, 0)),
        ],
        out_shape=[
            jax.ShapeDtypeStruct((NP, H), jnp.float32),
            jax.ShapeDtypeStruct((NP, H), jnp.float32),
        ],
    )(p, g, dinv, bf_row, Wa1, ba1_row, wa2_row, ba2_11, W1a, W1b)


def _bdot(a, b):
    return lax.dot_general(
        a.astype(jnp.bfloat16), b.astype(jnp.bfloat16),
        (((1,), (0,)), ((), ())), preferred_element_type=jnp.float32)


def _edge_kernel(s_ref, ea_ref, we1_ref, be1_ref, we2_ref, be2_ref,
                 wc1c_ref, bc1_ref, wc2_ref, bc2_ref, wc3_ref, bc3_ref,
                 out_ref):
    t = jnp.maximum(ea_ref[...] @ we1_ref[...] + be1_ref[...], 0.0)
    wprime = we2_ref[...] @ wc1c_ref[...]
    cprime = be2_ref[...] @ wc1c_ref[...] + bc1_ref[...]
    z = jnp.maximum(_bdot(t, wprime) + s_ref[...] + cprime, 0.0)
    z2 = jnp.maximum(_bdot(z, wc2_ref[...]) + bc2_ref[...], 0.0)
    logits = z2 @ wc3_ref[...] + bc3_ref[...]
    m = jnp.max(logits, axis=1, keepdims=True)
    lse = m + jnp.log(jnp.sum(jnp.exp(logits - m), axis=1, keepdims=True))
    out_ref[...] = logits - lse


def _edge_call(S, ea, We1, be1_row, We2, be2_row, Wc1c, bc1_row,
               Wc2, bc2_row, Wc3, bc3_row):
    full = lambda shape: pl.BlockSpec(shape, lambda i: tuple(0 for _ in shape))
    return pl.pallas_call(
        _edge_kernel,
        grid=(E // BE,),
        in_specs=[
            pl.BlockSpec((BE, H), lambda i: (i, 0)),
            pl.BlockSpec((BE, 4), lambda i: (i, 0)),
            full((4, H)),
            full((1, H)),
            full((H, H)),
            full((1, H)),
            full((H, H)),
            full((1, H)),
            full((H, H // 2)),
            full((1, H // 2)),
            full((H // 2, 2)),
            full((1, 2)),
        ],
        out_specs=pl.BlockSpec((BE, 2), lambda i: (i, 0)),
        out_shape=jax.ShapeDtypeStruct((E, 2), jnp.float32),
    )(S, ea, We1, be1_row, We2, be2_row, Wc1c, bc1_row,
      Wc2, bc2_row, Wc3, bc3_row)


def kernel(x, edge_index, edge_attr, W1, b1, W2, b2, W3, b3, Wf, bf,
           We1, be1, We2, be2, Wa1, ba1, Wa2, ba2,
           Wc1, bc1, Wc2, bc2, Wc3, bc3):
    src = edge_index[0]
    dst = edge_index[1]
    src2 = src.reshape(NW, NCHW, C)
    dst2 = dst.reshape(NW, NCHW, C)
    x_pad = jnp.pad(x, ((0, NP - N), (0, 0)))
    zeros_np = jnp.zeros((NP, H), jnp.float32)

    degp = _deg_call(dst)
    xw = _xw_call(x_pad, W1)
    dinv, g = _g1_call(degp, xw)

    for W_next, b_cur in ((W2, b1), (W3, b2), (Wf, b3)):
        p = _seg_call(g, src, dst, zeros_np)
        g = _comb_call(p, g, dinv, b_cur.reshape(1, H), W_next)

    p = _seg_call(g, src, dst, zeros_np)
    A, B = _post_call(
        p, g, dinv, bf.reshape(1, H), Wa1, ba1.reshape(1, H // 2),
        Wa2.reshape(1, H // 2), ba2.reshape(1, 1),
        Wc1[:H], Wc1[H:2 * H])

    S = _cls_call(A, B, src2, dst2)

    return _edge_call(
        S, edge_attr, We1, be1.reshape(1, H), We2, be2.reshape(1, H),
        Wc1[2 * H:], bc1.reshape(1, H), Wc2, bc2.reshape(1, H // 2),
        Wc3, bc3.reshape(1, 2))


# speedup vs baseline: 1.2366x; 1.0106x over previous
import functools

import jax
import jax.numpy as jnp
from jax import lax
from jax.experimental import pallas as pl
from jax.experimental.pallas import tpu as pltpu
from jax.experimental.pallas import tpu_sc as plsc

N = 10000
NP = 10240
E = 320000
H = 128
NC, NS = 2, 16
NW = NC * NS
C = 80
NCH = E // C
NCHW = NCH // NW
EPW = E // NW
NB = 4
H2 = H // 2
BN = 1024
BE = 12800

_mesh = plsc.VectorSubcoreMesh(
    core_axis_name="c", subcore_axis_name="s", num_cores=NC, num_subcores=NS)
_sc_params = pltpu.CompilerParams(needs_layout_passes=False)
_sc_params_lin = pltpu.CompilerParams(
    needs_layout_passes=False, use_tc_tiling_on_sc=False)


def _wid():
    return lax.axis_index("c") * NS + lax.axis_index("s")


def _deg_body(dst_hbm, out_hbm, deg_v, idx_v, sem):
    wid = _wid()
    idx_dma = pltpu.async_copy(dst_hbm.at[pl.ds(wid * EPW, EPW)], idx_v, sem)

    def zero(i, carry):
        deg_v[pl.ds(i * 16, 16)] = jnp.zeros((16,), jnp.float32)
        return carry
    lax.fori_loop(0, NP // 16, zero, 0)
    idx_dma.wait()

    ones = jnp.ones((16,), jnp.float32)

    def sub(j, carry):
        idx = idx_v[pl.ds(j * 16, 16)]
        plsc.addupdate_scatter(deg_v, [idx], ones)
        return carry
    lax.fori_loop(0, EPW // 16, sub, 0)
    pltpu.sync_copy(deg_v, out_hbm.at[wid])


_deg_call = pl.kernel(
    _deg_body,
    out_type=jax.ShapeDtypeStruct((NW, NP), jnp.float32),
    mesh=_mesh,
    compiler_params=_sc_params,
    scratch_types=[
        pltpu.VMEM((NP,), jnp.float32),
        pltpu.VMEM((EPW,), jnp.int32),
        pltpu.SemaphoreType.DMA,
    ],
)


NBS = 3


def _seg_body(g_hbm, src_hbm, dst_hbm, zero_hbm, p_hbm,
              idx_s, idx_d, rows, acc, gsem, ssem):
    cid = lax.axis_index("c")
    sid = lax.axis_index("s")
    wid = cid * NS + sid
    rpw = NP // NS

    pltpu.sync_copy(zero_hbm.at[pl.ds(sid * rpw, rpw)],
                    acc.at[pl.ds(sid * rpw, rpw)])
    plsc.subcore_barrier()

    def fire_gather(i, b):
        base = wid * EPW + i * C
        pltpu.sync_copy(src_hbm.at[pl.ds(base, C)], idx_s.at[b])
        pltpu.sync_copy(dst_hbm.at[pl.ds(base, C)], idx_d.at[b])
        pltpu.async_copy(g_hbm.at[idx_s.at[b]], rows.at[b], gsem)

    def wait_gather():
        pltpu.make_async_copy(g_hbm.at[idx_s.at[0]], rows.at[0], gsem).wait()

    def fire_scatter(b):
        pltpu.async_copy(rows.at[b], acc.at[idx_d.at[b]], ssem, add=True)

    def wait_scatter():
        pltpu.make_async_copy(rows.at[0], acc.at[idx_d.at[0]], ssem).wait()

    for i0 in range(NBS - 1):
        fire_gather(i0, i0)

    def grp(k, carry):
        for b in range(NBS):
            i = NBS * k + b

            @pl.when(i >= 1)
            def _():
                wait_scatter()

            @pl.when(i + NBS - 1 < NCHW)
            def _():
                fire_gather(i + NBS - 1, (b + NBS - 1) % NBS)
            wait_gather()
            fire_scatter(b)
        return carry
    lax.fori_loop(0, NCHW // NBS, grp, 0)
    for i in range(NCHW - NCHW % NBS, NCHW):
        wait_scatter()
        wait_gather()
        fire_scatter(i % NBS)
    wait_scatter()

    plsc.subcore_barrier()
    pltpu.sync_copy(acc.at[pl.ds(sid * rpw, rpw)],
                    p_hbm.at[cid].at[pl.ds(sid * rpw, rpw)])


_seg_call = pl.kernel(
    _seg_body,
    out_type=jax.ShapeDtypeStruct((NC, NP, H), jnp.float32),
    mesh=_mesh,
    compiler_params=_sc_params,
    scratch_types=[
        pltpu.VMEM((NBS, C), jnp.int32),
        pltpu.VMEM((NBS, C), jnp.int32),
        pltpu.VMEM((NBS, C, H), jnp.float32),
        pltpu.VMEM_SHARED((NP, H), jnp.float32),
        pltpu.SemaphoreType.DMA,
        pltpu.SemaphoreType.DMA,
    ],
)


def _cls_body(a_hbm, b_hbm, row2_hbm, col2_hbm, s_hbm,
              idx_r, idx_c, rows_a, rows_b, gsem, wsem):
    wid = _wid()
    pltpu.sync_copy(row2_hbm.at[wid], idx_r)
    pltpu.sync_copy(col2_hbm.at[wid], idx_c)

    def fire_gathers(i, b):
        pltpu.async_copy(a_hbm.at[idx_r.at[i]], rows_a.at[b], gsem)
        pltpu.async_copy(b_hbm.at[idx_c.at[i]], rows_b.at[b], gsem)

    def wait_gathers():
        pltpu.make_async_copy(a_hbm.at[idx_r.at[0]], rows_a.at[0], gsem).wait()
        pltpu.make_async_copy(b_hbm.at[idx_c.at[0]], rows_b.at[0], gsem).wait()

    def fire_store(i, b):
        base = (wid * NCHW + i) * C
        pltpu.async_copy(rows_a.at[b], s_hbm.at[pl.ds(base, C)], wsem)

    def wait_store():
        pltpu.make_async_copy(rows_a.at[0], s_hbm.at[pl.ds(0, C)], wsem).wait()

    def vadd(b):
        def add_row(r, c2):
            for col in range(H // 16):
                sl = pl.ds(col * 16, 16)
                rows_a[b, r, sl] = rows_a[b, r, sl] + rows_b[b, r, sl]
            return c2
        lax.fori_loop(0, C, add_row, 0)

    for i0 in range(NB - 1):
        fire_gathers(i0, i0)

    def quad(k, carry):
        for b in range(NB):
            i = NB * k + b

            @pl.when(i >= 1)
            def _():
                wait_store()

            @pl.when(i + NB - 1 < NCHW)
            def _():
                fire_gathers(i + NB - 1, (b + NB - 1) % NB)
            wait_gathers()
            vadd(b)
            fire_store(i, b)
        return carry
    lax.fori_loop(0, NCHW // NB, quad, 0)
    for i in range(NCHW - NCHW % NB, NCHW):
        wait_store()
        wait_gathers()
        vadd(i % NB)
        fire_store(i, i % NB)
    wait_store()


_cls_call = pl.kernel(
    _cls_body,
    out_type=jax.ShapeDtypeStruct((E, H), jnp.float32),
    mesh=_mesh,
    compiler_params=_sc_params,
    scratch_types=[
        pltpu.VMEM((NCHW, C), jnp.int32),
        pltpu.VMEM((NCHW, C), jnp.int32),
        pltpu.VMEM((NB, C, H), jnp.float32),
        pltpu.VMEM((NB, C, H), jnp.float32),
        pltpu.SemaphoreType.DMA,
        pltpu.SemaphoreType.DMA,
    ],
)


def _xw_kernel(x_ref, w_ref, out_ref):
    out_ref[...] = x_ref[...] @ w_ref[...]


def _xw_call(x_pad, W1):
    return pl.pallas_call(
        _xw_kernel,
        grid=(NP // BN,),
        in_specs=[
            pl.BlockSpec((BN, H), lambda i: (i, 0)),
            pl.BlockSpec((H, H), lambda i: (0, 0)),
        ],
        out_specs=pl.BlockSpec((BN, H), lambda i: (i, 0)),
        out_shape=jax.ShapeDtypeStruct((NP, H), jnp.float32),
    )(x_pad, W1)


def _g1_kernel(degp_ref, xw_ref, dinv_ref, g_ref):
    deg = jnp.sum(degp_ref[...], axis=0) + 1.0
    dinv = lax.rsqrt(deg)
    dinv_ref[...] = dinv
    g_ref[...] = xw_ref[...] * dinv[:, None]


def _g1_call(degp, xw):
    return pl.pallas_call(
        _g1_kernel,
        grid=(NP // BN,),
        in_specs=[
            pl.BlockSpec((NW, BN), lambda i: (0, i)),
            pl.BlockSpec((BN, H), lambda i: (i, 0)),
        ],
        out_specs=[
            pl.BlockSpec((BN,), lambda i: (i,)),
            pl.BlockSpec((BN, H), lambda i: (i, 0)),
        ],
        out_shape=[
            jax.ShapeDtypeStruct((NP,), jnp.float32),
            jax.ShapeDtypeStruct((NP, H), jnp.float32),
        ],
    )(degp, xw)


def _comb_kernel(p_ref, g_ref, dinv_ref, b_ref, w_ref, out_ref):
    dinv = dinv_ref[...]
    h = jnp.maximum(
        (p_ref[0] + p_ref[1] + g_ref[...]) * dinv[:, None] + b_ref[...], 0.0)
    out_ref[...] = (h @ w_ref[...]) * dinv[:, None]


def _comb_call(p, g, dinv, b_row, W_next):
    return pl.pallas_call(
        _comb_kernel,
        grid=(NP // BN,),
        in_specs=[
            pl.BlockSpec((NC, BN, H), lambda i: (0, i, 0)),
            pl.BlockSpec((BN, H), lambda i: (i, 0)),
            pl.BlockSpec((BN,), lambda i: (i,)),
            pl.BlockSpec((1, H), lambda i: (0, 0)),
            pl.BlockSpec((H, H), lambda i: (0, 0)),
        ],
        out_specs=pl.BlockSpec((BN, H), lambda i: (i, 0)),
        out_shape=jax.ShapeDtypeStruct((NP, H), jnp.float32),
    )(p, g, dinv, b_row, W_next)


def _post_kernel(p_ref, g_ref, dinv_ref, bf_ref, wa1_ref, ba1_ref,
                 wa2_ref, ba2_ref, w1a_ref, w1b_ref, a_ref, b_out_ref):
    dinv = dinv_ref[...]
    h = jnp.maximum(
        (p_ref[0] + p_ref[1] + g_ref[...]) * dinv[:, None] + bf_ref[...], 0.0)
    t = jnp.maximum(h @ wa1_ref[...] + ba1_ref[...], 0.0)
    att = jax.nn.sigmoid(
        jnp.sum(t * wa2_ref[...], axis=1, keepdims=True) + ba2_ref[...])
    hg = h * att
    a_ref[...] = hg @ w1a_ref[...]
    b_out_ref[...] = hg @ w1b_ref[...]


def _post_call(p, g, dinv, bf_row, Wa1, ba1_row, wa2_row, ba2_11, W1a, W1b):
    return pl.pallas_call(
        _post_kernel,
        grid=(NP // BN,),
        in_specs=[
            pl.BlockSpec((NC, BN, H), lambda i: (0, i, 0)),
            pl.BlockSpec((BN, H), lambda i: (i, 0)),
            pl.BlockSpec((BN,), lambda i: (i,)),
            pl.BlockSpec((1, H), lambda i: (0, 0)),
            pl.BlockSpec((H, H // 2), lambda i: (0, 0)),
            pl.BlockSpec((1, H // 2), lambda i: (0, 0)),
            pl.BlockSpec((1, H // 2), lambda i: (0, 0)),
            pl.BlockSpec((1, 1), lambda i: (0, 0)),
            pl.BlockSpec((H, H), lambda i: (0, 0)),
            pl.BlockSpec((H, H), lambda i: (0, 0)),
        ],
        out_specs=[
            pl.BlockSpec((BN, H), lambda i: (i, 0)),
            pl.BlockSpec((BN, H), lambda i: (i, 0)),
        ],
        out_shape=[
            jax.ShapeDtypeStruct((NP, H), jnp.float32),
            jax.ShapeDtypeStruct((NP, H), jnp.float32),
        ],
    )(p, g, dinv, bf_row, Wa1, ba1_row, wa2_row, ba2_11, W1a, W1b)


def _bdot(a, b):
    return lax.dot_general(
        a.astype(jnp.bfloat16), b.astype(jnp.bfloat16),
        (((1,), (0,)), ((), ())), preferred_element_type=jnp.float32)


def _edge_kernel(s_ref, ea_ref, we1_ref, be1_ref, we2_ref, be2_ref,
                 wc1c_ref, bc1_ref, wc2_ref, bc2_ref, wc3_ref, bc3_ref,
                 out_ref):
    t = jnp.maximum(ea_ref[...] @ we1_ref[...] + be1_ref[...], 0.0)
    wprime = we2_ref[...] @ wc1c_ref[...]
    cprime = be2_ref[...] @ wc1c_ref[...] + bc1_ref[...]
    z = jnp.maximum(_bdot(t, wprime) + s_ref[...] + cprime, 0.0)
    z2 = jnp.maximum(_bdot(z, wc2_ref[...]) + bc2_ref[...], 0.0)
    logits = z2 @ wc3_ref[...] + bc3_ref[...]
    m = jnp.max(logits, axis=1, keepdims=True)
    lse = m + jnp.log(jnp.sum(jnp.exp(logits - m), axis=1, keepdims=True))
    out_ref[...] = logits - lse


def _edge_call(S, ea, We1, be1_row, We2, be2_row, Wc1c, bc1_row,
               Wc2, bc2_row, Wc3, bc3_row):
    full = lambda shape: pl.BlockSpec(shape, lambda i: tuple(0 for _ in shape))
    return pl.pallas_call(
        _edge_kernel,
        grid=(E // BE,),
        in_specs=[
            pl.BlockSpec((BE, H), lambda i: (i, 0)),
            pl.BlockSpec((BE, 4), lambda i: (i, 0)),
            full((4, H)),
            full((1, H)),
            full((H, H)),
            full((1, H)),
            full((H, H)),
            full((1, H)),
            full((H, H // 2)),
            full((1, H // 2)),
            full((H // 2, 2)),
            full((1, 2)),
        ],
        out_specs=pl.BlockSpec((BE, 2), lambda i: (i, 0)),
        out_shape=jax.ShapeDtypeStruct((E, 2), jnp.float32),
    )(S, ea, We1, be1_row, We2, be2_row, Wc1c, bc1_row,
      Wc2, bc2_row, Wc3, bc3_row)


def kernel(x, edge_index, edge_attr, W1, b1, W2, b2, W3, b3, Wf, bf,
           We1, be1, We2, be2, Wa1, ba1, Wa2, ba2,
           Wc1, bc1, Wc2, bc2, Wc3, bc3):
    src = edge_index[0]
    dst = edge_index[1]
    src2 = src.reshape(NW, NCHW, C)
    dst2 = dst.reshape(NW, NCHW, C)
    x_pad = jnp.pad(x, ((0, NP - N), (0, 0)))
    zeros_np = jnp.zeros((NP, H), jnp.float32)

    degp = _deg_call(dst)
    xw = _xw_call(x_pad, W1)
    dinv, g = _g1_call(degp, xw)

    for W_next, b_cur in ((W2, b1), (W3, b2), (Wf, b3)):
        p = _seg_call(g, src, dst, zeros_np)
        g = _comb_call(p, g, dinv, b_cur.reshape(1, H), W_next)

    p = _seg_call(g, src, dst, zeros_np)
    A, B = _post_call(
        p, g, dinv, bf.reshape(1, H), Wa1, ba1.reshape(1, H // 2),
        Wa2.reshape(1, H // 2), ba2.reshape(1, 1),
        Wc1[:H], Wc1[H:2 * H])

    S = _cls_call(A, B, src2, dst2)

    return _edge_call(
        S, edge_attr, We1, be1.reshape(1, H), We2, be2.reshape(1, H),
        Wc1[2 * H:], bc1.reshape(1, H), Wc2, bc2.reshape(1, H // 2),
        Wc3, bc3.reshape(1, 2))


# speedup vs baseline: 1.2408x; 1.0034x over previous
import functools

import jax
import jax.numpy as jnp
from jax import lax
from jax.experimental import pallas as pl
from jax.experimental.pallas import tpu as pltpu
from jax.experimental.pallas import tpu_sc as plsc

N = 10000
NP = 10240
E = 320000
H = 128
NC, NS = 2, 16
NW = NC * NS
C = 80
NCH = E // C
NCHW = NCH // NW
EPW = E // NW
NB = 4
H2 = H // 2
BN = 1024
BE = 16000

_mesh = plsc.VectorSubcoreMesh(
    core_axis_name="c", subcore_axis_name="s", num_cores=NC, num_subcores=NS)
_sc_params = pltpu.CompilerParams(needs_layout_passes=False)
_sc_params_lin = pltpu.CompilerParams(
    needs_layout_passes=False, use_tc_tiling_on_sc=False)


def _wid():
    return lax.axis_index("c") * NS + lax.axis_index("s")


def _deg_body(dst_hbm, out_hbm, deg_v, idx_v, sem):
    wid = _wid()
    idx_dma = pltpu.async_copy(dst_hbm.at[pl.ds(wid * EPW, EPW)], idx_v, sem)

    def zero(i, carry):
        deg_v[pl.ds(i * 16, 16)] = jnp.zeros((16,), jnp.float32)
        return carry
    lax.fori_loop(0, NP // 16, zero, 0)
    idx_dma.wait()

    ones = jnp.ones((16,), jnp.float32)

    def sub(j, carry):
        idx = idx_v[pl.ds(j * 16, 16)]
        plsc.addupdate_scatter(deg_v, [idx], ones)
        return carry
    lax.fori_loop(0, EPW // 16, sub, 0)
    pltpu.sync_copy(deg_v, out_hbm.at[wid])


_deg_call = pl.kernel(
    _deg_body,
    out_type=jax.ShapeDtypeStruct((NW, NP), jnp.float32),
    mesh=_mesh,
    compiler_params=_sc_params,
    scratch_types=[
        pltpu.VMEM((NP,), jnp.float32),
        pltpu.VMEM((EPW,), jnp.int32),
        pltpu.SemaphoreType.DMA,
    ],
)


NBS = 3


def _seg_body(g_hbm, src_hbm, dst_hbm, zero_hbm, p_hbm,
              idx_s, idx_d, rows, acc, gsem, ssem):
    cid = lax.axis_index("c")
    sid = lax.axis_index("s")
    wid = cid * NS + sid
    rpw = NP // NS

    pltpu.sync_copy(zero_hbm.at[pl.ds(sid * rpw, rpw)],
                    acc.at[pl.ds(sid * rpw, rpw)])
    plsc.subcore_barrier()

    def fire_gather(i, b):
        base = wid * EPW + i * C
        pltpu.sync_copy(src_hbm.at[pl.ds(base, C)], idx_s.at[b])
        pltpu.sync_copy(dst_hbm.at[pl.ds(base, C)], idx_d.at[b])
        pltpu.async_copy(g_hbm.at[idx_s.at[b]], rows.at[b], gsem)

    def wait_gather():
        pltpu.make_async_copy(g_hbm.at[idx_s.at[0]], rows.at[0], gsem).wait()

    def fire_scatter(b):
        pltpu.async_copy(rows.at[b], acc.at[idx_d.at[b]], ssem, add=True)

    def wait_scatter():
        pltpu.make_async_copy(rows.at[0], acc.at[idx_d.at[0]], ssem).wait()

    for i0 in range(NBS - 1):
        fire_gather(i0, i0)

    def grp(k, carry):
        for b in range(NBS):
            i = NBS * k + b

            @pl.when(i >= 1)
            def _():
                wait_scatter()

            @pl.when(i + NBS - 1 < NCHW)
            def _():
                fire_gather(i + NBS - 1, (b + NBS - 1) % NBS)
            wait_gather()
            fire_scatter(b)
        return carry
    lax.fori_loop(0, NCHW // NBS, grp, 0)
    for i in range(NCHW - NCHW % NBS, NCHW):
        wait_scatter()
        wait_gather()
        fire_scatter(i % NBS)
    wait_scatter()

    plsc.subcore_barrier()
    pltpu.sync_copy(acc.at[pl.ds(sid * rpw, rpw)],
                    p_hbm.at[cid].at[pl.ds(sid * rpw, rpw)])


_seg_call = pl.kernel(
    _seg_body,
    out_type=jax.ShapeDtypeStruct((NC, NP, H), jnp.float32),
    mesh=_mesh,
    compiler_params=_sc_params,
    scratch_types=[
        pltpu.VMEM((NBS, C), jnp.int32),
        pltpu.VMEM((NBS, C), jnp.int32),
        pltpu.VMEM((NBS, C, H), jnp.float32),
        pltpu.VMEM_SHARED((NP, H), jnp.float32),
        pltpu.SemaphoreType.DMA,
        pltpu.SemaphoreType.DMA,
    ],
)


def _cls_body(a_hbm, b_hbm, row2_hbm, col2_hbm, s_hbm,
              idx_r, idx_c, rows_a, rows_b, gsem, wsem):
    wid = _wid()
    pltpu.sync_copy(row2_hbm.at[wid], idx_r)
    pltpu.sync_copy(col2_hbm.at[wid], idx_c)

    def fire_gathers(i, b):
        pltpu.async_copy(a_hbm.at[idx_r.at[i]], rows_a.at[b], gsem)
        pltpu.async_copy(b_hbm.at[idx_c.at[i]], rows_b.at[b], gsem)

    def wait_gathers():
        pltpu.make_async_copy(a_hbm.at[idx_r.at[0]], rows_a.at[0], gsem).wait()
        pltpu.make_async_copy(b_hbm.at[idx_c.at[0]], rows_b.at[0], gsem).wait()

    def fire_store(i, b):
        base = (wid * NCHW + i) * C
        pltpu.async_copy(rows_a.at[b], s_hbm.at[pl.ds(base, C)], wsem)

    def wait_store():
        pltpu.make_async_copy(rows_a.at[0], s_hbm.at[pl.ds(0, C)], wsem).wait()

    def vadd(b):
        def add_row(r, c2):
            for col in range(H // 16):
                sl = pl.ds(col * 16, 16)
                rows_a[b, r, sl] = rows_a[b, r, sl] + rows_b[b, r, sl]
            return c2
        lax.fori_loop(0, C, add_row, 0)

    for i0 in range(NB - 1):
        fire_gathers(i0, i0)

    def quad(k, carry):
        for b in range(NB):
            i = NB * k + b

            @pl.when(i >= 1)
            def _():
                wait_store()

            @pl.when(i + NB - 1 < NCHW)
            def _():
                fire_gathers(i + NB - 1, (b + NB - 1) % NB)
            wait_gathers()
            vadd(b)
            fire_store(i, b)
        return carry
    lax.fori_loop(0, NCHW // NB, quad, 0)
    for i in range(NCHW - NCHW % NB, NCHW):
        wait_store()
        wait_gathers()
        vadd(i % NB)
        fire_store(i, i % NB)
    wait_store()


_cls_call = pl.kernel(
    _cls_body,
    out_type=jax.ShapeDtypeStruct((E, H), jnp.float32),
    mesh=_mesh,
    compiler_params=_sc_params,
    scratch_types=[
        pltpu.VMEM((NCHW, C), jnp.int32),
        pltpu.VMEM((NCHW, C), jnp.int32),
        pltpu.VMEM((NB, C, H), jnp.float32),
        pltpu.VMEM((NB, C, H), jnp.float32),
        pltpu.SemaphoreType.DMA,
        pltpu.SemaphoreType.DMA,
    ],
)


def _xw_kernel(x_ref, w_ref, out_ref):
    out_ref[...] = x_ref[...] @ w_ref[...]


def _xw_call(x_pad, W1):
    return pl.pallas_call(
        _xw_kernel,
        grid=(NP // BN,),
        in_specs=[
            pl.BlockSpec((BN, H), lambda i: (i, 0)),
            pl.BlockSpec((H, H), lambda i: (0, 0)),
        ],
        out_specs=pl.BlockSpec((BN, H), lambda i: (i, 0)),
        out_shape=jax.ShapeDtypeStruct((NP, H), jnp.float32),
    )(x_pad, W1)


def _g1_kernel(degp_ref, xw_ref, dinv_ref, g_ref):
    deg = jnp.sum(degp_ref[...], axis=0) + 1.0
    dinv = lax.rsqrt(deg)
    dinv_ref[...] = dinv
    g_ref[...] = xw_ref[...] * dinv[:, None]


def _g1_call(degp, xw):
    return pl.pallas_call(
        _g1_kernel,
        grid=(NP // BN,),
        in_specs=[
            pl.BlockSpec((NW, BN), lambda i: (0, i)),
            pl.BlockSpec((BN, H), lambda i: (i, 0)),
        ],
        out_specs=[
            pl.BlockSpec((BN,), lambda i: (i,)),
            pl.BlockSpec((BN, H), lambda i: (i, 0)),
        ],
        out_shape=[
            jax.ShapeDtypeStruct((NP,), jnp.float32),
            jax.ShapeDtypeStruct((NP, H), jnp.float32),
        ],
    )(degp, xw)


def _comb_kernel(p_ref, g_ref, dinv_ref, b_ref, w_ref, out_ref):
    dinv = dinv_ref[...]
    h = jnp.maximum(
        (p_ref[0] + p_ref[1] + g_ref[...]) * dinv[:, None] + b_ref[...], 0.0)
    out_ref[...] = (h @ w_ref[...]) * dinv[:, None]


def _comb_call(p, g, dinv, b_row, W_next):
    return pl.pallas_call(
        _comb_kernel,
        grid=(NP // BN,),
        in_specs=[
            pl.BlockSpec((NC, BN, H), lambda i: (0, i, 0)),
            pl.BlockSpec((BN, H), lambda i: (i, 0)),
            pl.BlockSpec((BN,), lambda i: (i,)),
            pl.BlockSpec((1, H), lambda i: (0, 0)),
            pl.BlockSpec((H, H), lambda i: (0, 0)),
        ],
        out_specs=pl.BlockSpec((BN, H), lambda i: (i, 0)),
        out_shape=jax.ShapeDtypeStruct((NP, H), jnp.float32),
    )(p, g, dinv, b_row, W_next)


def _post_kernel(p_ref, g_ref, dinv_ref, bf_ref, wa1_ref, ba1_ref,
                 wa2_ref, ba2_ref, w1a_ref, w1b_ref, a_ref, b_out_ref):
    dinv = dinv_ref[...]
    h = jnp.maximum(
        (p_ref[0] + p_ref[1] + g_ref[...]) * dinv[:, None] + bf_ref[...], 0.0)
    t = jnp.maximum(h @ wa1_ref[...] + ba1_ref[...], 0.0)
    att = jax.nn.sigmoid(
        jnp.sum(t * wa2_ref[...], axis=1, keepdims=True) + ba2_ref[...])
    hg = h * att
    a_ref[...] = hg @ w1a_ref[...]
    b_out_ref[...] = hg @ w1b_ref[...]


def _post_call(p, g, dinv, bf_row, Wa1, ba1_row, wa2_row, ba2_11, W1a, W1b):
    return pl.pallas_call(
        _post_kernel,
        grid=(NP // BN,),
        in_specs=[
            pl.BlockSpec((NC, BN, H), lambda i: (0, i, 0)),
            pl.BlockSpec((BN, H), lambda i: (i, 0)),
            pl.BlockSpec((BN,), lambda i: (i,)),
            pl.BlockSpec((1, H), lambda i: (0, 0)),
            pl.BlockSpec((H, H // 2), lambda i: (0, 0)),
            pl.BlockSpec((1, H // 2), lambda i: (0, 0)),
            pl.BlockSpec((1, H // 2), lambda i: (0, 0)),
            pl.BlockSpec((1, 1), lambda i: (0, 0)),
            pl.BlockSpec((H, H), lambda i: (0, 0)),
            pl.BlockSpec((H, H), lambda i: (0, 0)),
        ],
        out_specs=[
            pl.BlockSpec((BN, H), lambda i: (i, 0)),
            pl.BlockSpec((BN, H), lambda i: (i, 0)),
        ],
        out_shape=[
            jax.ShapeDtypeStruct((NP, H), jnp.float32),
            jax.ShapeDtypeStruct((NP, H), jnp.float32),
        ],
    )(p, g, dinv, bf_row, Wa1, ba1_row, wa2_row, ba2_11, W1a, W1b)


def _bdot(a, b):
    return lax.dot_general(
        a.astype(jnp.bfloat16), b.astype(jnp.bfloat16),
        (((1,), (0,)), ((), ())), preferred_element_type=jnp.float32)


def _edge_kernel(s_ref, ea_ref, we1_ref, be1_ref, we2_ref, be2_ref,
                 wc1c_ref, bc1_ref, wc2_ref, bc2_ref, wc3_ref, bc3_ref,
                 out_ref):
    t = jnp.maximum(ea_ref[...] @ we1_ref[...] + be1_ref[...], 0.0)
    wprime = we2_ref[...] @ wc1c_ref[...]
    cprime = be2_ref[...] @ wc1c_ref[...] + bc1_ref[...]
    z = jnp.maximum(_bdot(t, wprime) + s_ref[...] + cprime, 0.0)
    z2 = jnp.maximum(_bdot(z, wc2_ref[...]) + bc2_ref[...], 0.0)
    logits = z2 @ wc3_ref[...] + bc3_ref[...]
    m = jnp.max(logits, axis=1, keepdims=True)
    lse = m + jnp.log(jnp.sum(jnp.exp(logits - m), axis=1, keepdims=True))
    out_ref[...] = logits - lse


def _edge_call(S, ea, We1, be1_row, We2, be2_row, Wc1c, bc1_row,
               Wc2, bc2_row, Wc3, bc3_row):
    full = lambda shape: pl.BlockSpec(shape, lambda i: tuple(0 for _ in shape))
    return pl.pallas_call(
        _edge_kernel,
        grid=(E // BE,),
        in_specs=[
            pl.BlockSpec((BE, H), lambda i: (i, 0)),
            pl.BlockSpec((BE, 4), lambda i: (i, 0)),
            full((4, H)),
            full((1, H)),
            full((H, H)),
            full((1, H)),
            full((H, H)),
            full((1, H)),
            full((H, H // 2)),
            full((1, H // 2)),
            full((H // 2, 2)),
            full((1, 2)),
        ],
        out_specs=pl.BlockSpec((BE, 2), lambda i: (i, 0)),
        out_shape=jax.ShapeDtypeStruct((E, 2), jnp.float32),
    )(S, ea, We1, be1_row, We2, be2_row, Wc1c, bc1_row,
      Wc2, bc2_row, Wc3, bc3_row)


def kernel(x, edge_index, edge_attr, W1, b1, W2, b2, W3, b3, Wf, bf,
           We1, be1, We2, be2, Wa1, ba1, Wa2, ba2,
           Wc1, bc1, Wc2, bc2, Wc3, bc3):
    src = edge_index[0]
    dst = edge_index[1]
    src2 = src.reshape(NW, NCHW, C)
    dst2 = dst.reshape(NW, NCHW, C)
    x_pad = jnp.pad(x, ((0, NP - N), (0, 0)))
    zeros_np = jnp.zeros((NP, H), jnp.float32)

    degp = _deg_call(dst)
    xw = _xw_call(x_pad, W1)
    dinv, g = _g1_call(degp, xw)

    for W_next, b_cur in ((W2, b1), (W3, b2), (Wf, b3)):
        p = _seg_call(g, src, dst, zeros_np)
        g = _comb_call(p, g, dinv, b_cur.reshape(1, H), W_next)

    p = _seg_call(g, src, dst, zeros_np)
    A, B = _post_call(
        p, g, dinv, bf.reshape(1, H), Wa1, ba1.reshape(1, H // 2),
        Wa2.reshape(1, H // 2), ba2.reshape(1, 1),
        Wc1[:H], Wc1[H:2 * H])

    S = _cls_call(A, B, src2, dst2)

    return _edge_call(
        S, edge_attr, We1, be1.reshape(1, H), We2, be2.reshape(1, H),
        Wc1[2 * H:], bc1.reshape(1, H), Wc2, bc2.reshape(1, H // 2),
        Wc3, bc3.reshape(1, 2))
